# baseline (speedup 1.0000x reference)
import jax
import jax.numpy as jnp
from jax import lax
from jax.experimental import pallas as pl
from jax.experimental.pallas import tpu as pltpu
from jax.experimental.pallas import tpu_sc as plsc

_VOCAB = 100000 + 1
_EMBED_DIM = 400
_L = 128
_K = 4
_VB = 2048
_VPAD = ((_VOCAB + _VB - 1) // _VB) * _VB

_INFO = plsc.get_sparse_core_info()
_NC = _INFO.num_cores
_NS = _INFO.num_subcores
_NW = _NC * _NS

_R = 4096
_S = 50
_NROW = _R * _S * _K
_ROW_PER_W = _NROW // _NW
_CHUNK = 128
_NCHUNK = _ROW_PER_W // _CHUNK
_BC = 128


def _tc_prep_body(tab_t_ref, out_ref):
    out_ref[...] = jnp.transpose(tab_t_ref[...], (1, 0))


def _tc_prep(tab_t):
    return pl.pallas_call(
        _tc_prep_body,
        grid=(_K, _VPAD // _VB),
        in_specs=[pl.BlockSpec((_L, _VB), lambda k, i: (k, i))],
        out_specs=pl.BlockSpec(
            (_VB, _L), lambda k, i: (k * (_VPAD // _VB) + i, 0)
        ),
        out_shape=jax.ShapeDtypeStruct((_K * _VPAD, _L), jnp.float32),
    )(tab_t)


def _sc_body(x4_hbm, tab4_hbm, mid_hbm, idx_v, buf0, buf1, sg0, sg1, sw0, sw1):
    wid = lax.axis_index("s") * _NC + lax.axis_index("c")
    base = wid * _ROW_PER_W
    pltpu.sync_copy(x4_hbm.at[pl.ds(base, _ROW_PER_W)], idx_v)

    bufs = (buf0, buf1)
    sgs = (sg0, sg1)
    sws = (sw0, sw1)

    @pl.loop(0, _NCHUNK, step=2)
    def _chunks(g):
        for b in range(2):
            cc = g + b

            @pl.when(cc >= 2)
            def _drain_write():
                pltpu.make_async_copy(
                    bufs[b], mid_hbm.at[pl.ds(base, _CHUNK)], sws[b]
                ).wait()

            pltpu.async_copy(
                tab4_hbm.at[idx_v.at[pl.ds(cc * _CHUNK, _CHUNK)]],
                bufs[b],
                sgs[b],
            )
            pltpu.make_async_copy(
                tab4_hbm.at[idx_v.at[pl.ds(0, _CHUNK)]], bufs[b], sgs[b]
            ).wait()
            pltpu.async_copy(
                bufs[b], mid_hbm.at[pl.ds(base + cc * _CHUNK, _CHUNK)], sws[b]
            )

    for b in range(2):
        pltpu.make_async_copy(
            bufs[b], mid_hbm.at[pl.ds(base, _CHUNK)], sws[b]
        ).wait()


def _sc_gather(x4, tab4):
    run = pl.kernel(
        _sc_body,
        out_type=jax.ShapeDtypeStruct((_NROW, _L), jnp.float32),
        mesh=plsc.VectorSubcoreMesh(core_axis_name="c", subcore_axis_name="s"),
        scratch_types=[
            pltpu.VMEM((_ROW_PER_W,), jnp.int32),
            pltpu.VMEM((_CHUNK, _L), jnp.float32),
            pltpu.VMEM((_CHUNK, _L), jnp.float32),
            pltpu.SemaphoreType.DMA,
            pltpu.SemaphoreType.DMA,
            pltpu.SemaphoreType.DMA,
            pltpu.SemaphoreType.DMA,
        ],
    )
    return run(x4, tab4)


def _tc_out_body(mid_ref, out_ref):
    x = mid_ref[...]
    for s in range(_S):
        for k in range(_K):
            t = jnp.transpose(x[k, :, s, :], (1, 0))
            lo = _L * k
            hi = min(_L * (k + 1), _EMBED_DIM)
            out_ref[s, lo:hi, :] = t[: hi - lo, :]


def _tc_out(mid4):
    return pl.pallas_call(
        _tc_out_body,
        grid=(_R // _BC,),
        in_specs=[pl.BlockSpec((_K, _BC, _S, _L), lambda i: (0, i, 0, 0))],
        out_specs=pl.BlockSpec((_S, _EMBED_DIM, _BC), lambda i: (0, 0, i)),
        out_shape=jax.ShapeDtypeStruct((_S, _EMBED_DIM, _R), jnp.float32),
        compiler_params=pltpu.CompilerParams(
            vmem_limit_bytes=56 * 1024 * 1024
        ),
    )(mid4)


def kernel(x, table):
    tab4 = _tc_prep(jnp.transpose(table, (1, 0)))
    x_flat = x.reshape(-1).astype(jnp.int32)
    x4 = (
        x_flat[None, :] + (jnp.arange(_K, dtype=jnp.int32) * _VPAD)[:, None]
    ).reshape(-1)
    mid = _sc_gather(x4, tab4)
    out_t = _tc_out(mid.reshape(_K, _R, _S, _L))
    return jnp.transpose(out_t, (2, 0, 1))


# speedup vs baseline: 1.3344x; 1.3344x over previous
import jax
import jax.numpy as jnp
from jax import lax
from jax.experimental import pallas as pl
from jax.experimental.pallas import tpu as pltpu
from jax.experimental.pallas import tpu_sc as plsc

_VOCAB = 100000 + 1
_EMBED_DIM = 400
_L = 128
_K = 4
_VB = 2048
_VPAD = ((_VOCAB + _VB - 1) // _VB) * _VB

_INFO = plsc.get_sparse_core_info()
_NC = _INFO.num_cores
_NS = _INFO.num_subcores
_NW = _NC * _NS

_R = 4096
_S = 50
_NROW = _R * _S * _K
_ROW_PER_W = _NROW // _NW
_CHUNK = 128
_NCHUNK = _ROW_PER_W // _CHUNK
_BC = 128


def _tc_prep_body(tab_t_ref, out_ref):
    out_ref[...] = jnp.transpose(tab_t_ref[...], (1, 0))


def _tc_prep(tab_t):
    return pl.pallas_call(
        _tc_prep_body,
        grid=(_K, _VPAD // _VB),
        in_specs=[pl.BlockSpec((_L, _VB), lambda k, i: (k, i))],
        out_specs=pl.BlockSpec(
            (_VB, _L), lambda k, i: (k * (_VPAD // _VB) + i, 0)
        ),
        out_shape=jax.ShapeDtypeStruct((_K * _VPAD, _L), jnp.float32),
    )(tab_t)


def _sc_body(x4_hbm, tab4_hbm, mid_hbm, idx_v, buf0, buf1, sg0, sg1, sw0, sw1):
    wid = lax.axis_index("s") * _NC + lax.axis_index("c")
    base = wid * _ROW_PER_W
    pltpu.sync_copy(x4_hbm.at[pl.ds(base, _ROW_PER_W)], idx_v)

    bufs = (buf0, buf1)
    sgs = (sg0, sg1)
    sws = (sw0, sw1)

    @pl.loop(0, _NCHUNK, step=2)
    def _chunks(g):
        for b in range(2):
            cc = g + b

            @pl.when(cc >= 2)
            def _drain_write():
                pltpu.make_async_copy(
                    bufs[b], mid_hbm.at[pl.ds(base, _CHUNK)], sws[b]
                ).wait()

            pltpu.async_copy(
                tab4_hbm.at[idx_v.at[pl.ds(cc * _CHUNK, _CHUNK)]],
                bufs[b],
                sgs[b],
            )
            pltpu.make_async_copy(
                tab4_hbm.at[idx_v.at[pl.ds(0, _CHUNK)]], bufs[b], sgs[b]
            ).wait()
            pltpu.async_copy(
                bufs[b], mid_hbm.at[pl.ds(base + cc * _CHUNK, _CHUNK)], sws[b]
            )

    for b in range(2):
        pltpu.make_async_copy(
            bufs[b], mid_hbm.at[pl.ds(base, _CHUNK)], sws[b]
        ).wait()


def _sc_gather(x4, tab4):
    run = pl.kernel(
        _sc_body,
        out_type=jax.ShapeDtypeStruct((_NROW, _L), jnp.float32),
        mesh=plsc.VectorSubcoreMesh(core_axis_name="c", subcore_axis_name="s"),
        scratch_types=[
            pltpu.VMEM((_ROW_PER_W,), jnp.int32),
            pltpu.VMEM((_CHUNK, _L), jnp.float32),
            pltpu.VMEM((_CHUNK, _L), jnp.float32),
            pltpu.SemaphoreType.DMA,
            pltpu.SemaphoreType.DMA,
            pltpu.SemaphoreType.DMA,
            pltpu.SemaphoreType.DMA,
        ],
    )
    return run(x4, tab4)


def _tc_out_body(mid_ref, out_ref):
    x = mid_ref[...]
    for s in range(_S):
        for k in range(_K):
            t = jnp.transpose(x[:, _K * s + k, :], (1, 0))
            lo = _L * k
            hi = min(_L * (k + 1), _EMBED_DIM)
            out_ref[s, lo:hi, :] = t[: hi - lo, :]


def _tc_out(mid4):
    return pl.pallas_call(
        _tc_out_body,
        grid=(_R // _BC,),
        in_specs=[pl.BlockSpec((_BC, _S * _K, _L), lambda i: (i, 0, 0))],
        out_specs=pl.BlockSpec((_S, _EMBED_DIM, _BC), lambda i: (0, 0, i)),
        out_shape=jax.ShapeDtypeStruct((_S, _EMBED_DIM, _R), jnp.float32),
        compiler_params=pltpu.CompilerParams(
            vmem_limit_bytes=56 * 1024 * 1024
        ),
    )(mid4)


def kernel(x, table):
    tab4 = _tc_prep(jnp.transpose(table, (1, 0)))
    x_flat = x.reshape(-1).astype(jnp.int32)
    xr = jnp.repeat(x_flat.reshape(-1, _L // _K), _K, axis=1)
    kpat = jnp.tile(jnp.arange(_K, dtype=jnp.int32) * _VPAD, _L // _K)
    x4 = (xr + kpat[None, :]).reshape(-1)
    mid = _sc_gather(x4, tab4)
    out_t = _tc_out(mid.reshape(_R, _S * _K, _L))
    return jnp.transpose(out_t, (2, 0, 1))


# speedup vs baseline: 1.4273x; 1.0697x over previous
import jax
import jax.numpy as jnp
from jax import lax
from jax.experimental import pallas as pl
from jax.experimental.pallas import tpu as pltpu
from jax.experimental.pallas import tpu_sc as plsc

_VOCAB = 100000 + 1
_EMBED_DIM = 400
_L = 128
_K = 4
_VB = 2048
_VPAD = ((_VOCAB + _VB - 1) // _VB) * _VB

_INFO = plsc.get_sparse_core_info()
_NC = _INFO.num_cores
_NS = _INFO.num_subcores
_NW = _NC * _NS

_R = 4096
_S = 50
_NCH = 4
_RC = _R // _NCH
_NROW = _RC * _S * _K
_ROW_PER_W = _NROW // _NW
_CHUNK = 128
_NCHUNK = _ROW_PER_W // _CHUNK
_BC = 128


def _tc_prep_body(tab_t_ref, out_ref):
    out_ref[...] = jnp.transpose(tab_t_ref[...], (1, 0))


def _tc_prep(tab_t):
    return pl.pallas_call(
        _tc_prep_body,
        grid=(_K, _VPAD // _VB),
        in_specs=[pl.BlockSpec((_L, _VB), lambda k, i: (k, i))],
        out_specs=pl.BlockSpec(
            (_VB, _L), lambda k, i: (k * (_VPAD // _VB) + i, 0)
        ),
        out_shape=jax.ShapeDtypeStruct((_K * _VPAD, _L), jnp.float32),
    )(tab_t)


def _sc_body(x4_hbm, tab4_hbm, mid_hbm, idx_v, buf0, buf1, sg0, sg1, sw0, sw1):
    wid = lax.axis_index("s") * _NC + lax.axis_index("c")
    base = wid * _ROW_PER_W
    pltpu.sync_copy(x4_hbm.at[pl.ds(base, _ROW_PER_W)], idx_v)

    bufs = (buf0, buf1)
    sgs = (sg0, sg1)
    sws = (sw0, sw1)

    @pl.loop(0, _NCHUNK, step=2)
    def _chunks(g):
        for b in range(2):
            cc = g + b

            @pl.when(cc >= 2)
            def _drain_write():
                pltpu.make_async_copy(
                    bufs[b], mid_hbm.at[pl.ds(base, _CHUNK)], sws[b]
                ).wait()

            pltpu.async_copy(
                tab4_hbm.at[idx_v.at[pl.ds(cc * _CHUNK, _CHUNK)]],
                bufs[b],
                sgs[b],
            )
            pltpu.make_async_copy(
                tab4_hbm.at[idx_v.at[pl.ds(0, _CHUNK)]], bufs[b], sgs[b]
            ).wait()
            pltpu.async_copy(
                bufs[b], mid_hbm.at[pl.ds(base + cc * _CHUNK, _CHUNK)], sws[b]
            )

    for b in range(2):
        pltpu.make_async_copy(
            bufs[b], mid_hbm.at[pl.ds(base, _CHUNK)], sws[b]
        ).wait()


def _sc_gather(x4, tab4):
    run = pl.kernel(
        _sc_body,
        out_type=jax.ShapeDtypeStruct((_NROW, _L), jnp.float32),
        mesh=plsc.VectorSubcoreMesh(core_axis_name="c", subcore_axis_name="s"),
        scratch_types=[
            pltpu.VMEM((_ROW_PER_W,), jnp.int32),
            pltpu.VMEM((_CHUNK, _L), jnp.float32),
            pltpu.VMEM((_CHUNK, _L), jnp.float32),
            pltpu.SemaphoreType.DMA,
            pltpu.SemaphoreType.DMA,
            pltpu.SemaphoreType.DMA,
            pltpu.SemaphoreType.DMA,
        ],
    )
    return run(x4, tab4)


def _tc_out_body(mid_ref, out_ref):
    x = mid_ref[...]
    for s in range(_S):
        for k in range(_K):
            t = jnp.transpose(x[:, _K * s + k, :], (1, 0))
            lo = _L * k
            hi = min(_L * (k + 1), _EMBED_DIM)
            out_ref[s, lo:hi, :] = t[: hi - lo, :]


def _tc_out_acc_body(mid_ref, acc_ref, out_ref):
    del acc_ref
    _tc_out_body(mid_ref, out_ref)


def _tc_out(mid4, c, acc):
    off = c * (_RC // _BC)
    out_shape = jax.ShapeDtypeStruct((_S, _EMBED_DIM, _R), jnp.float32)
    params = pltpu.CompilerParams(vmem_limit_bytes=56 * 1024 * 1024)
    mid_spec = pl.BlockSpec((_BC, _S * _K, _L), lambda i: (i, 0, 0))
    out_spec = pl.BlockSpec(
        (_S, _EMBED_DIM, _BC), lambda i, off=off: (0, 0, off + i)
    )
    if acc is None:
        return pl.pallas_call(
            _tc_out_body,
            grid=(_RC // _BC,),
            in_specs=[mid_spec],
            out_specs=out_spec,
            out_shape=out_shape,
            compiler_params=params,
        )(mid4)
    return pl.pallas_call(
        _tc_out_acc_body,
        grid=(_RC // _BC,),
        in_specs=[mid_spec, pl.BlockSpec(memory_space=pl.ANY)],
        out_specs=out_spec,
        out_shape=out_shape,
        input_output_aliases={1: 0},
        compiler_params=params,
    )(mid4, acc)


def kernel(x, table):
    tab4 = _tc_prep(jnp.transpose(table, (1, 0)))
    x_flat = x.reshape(-1).astype(jnp.int32)
    xr = jnp.repeat(x_flat.reshape(-1, _L // _K), _K, axis=1)
    kpat = jnp.tile(jnp.arange(_K, dtype=jnp.int32) * _VPAD, _L // _K)
    x4 = (xr + kpat[None, :]).reshape(-1)
    acc = None
    for c in range(_NCH):
        x4_c = lax.slice(x4, (c * _NROW,), ((c + 1) * _NROW,))
        mid = _sc_gather(x4_c, tab4)
        acc = _tc_out(mid.reshape(_RC, _S * _K, _L), c, acc)
    return jnp.transpose(acc, (2, 0, 1))


# speedup vs baseline: 1.4989x; 1.0501x over previous
import jax
import jax.numpy as jnp
from jax import lax
from jax.experimental import pallas as pl
from jax.experimental.pallas import tpu as pltpu
from jax.experimental.pallas import tpu_sc as plsc

_VOCAB = 100000 + 1
_EMBED_DIM = 400
_L = 128
_K = 4
_VB = 2048
_VPAD = ((_VOCAB + _VB - 1) // _VB) * _VB

_INFO = plsc.get_sparse_core_info()
_NC = _INFO.num_cores
_NS = _INFO.num_subcores
_NW = _NC * _NS

_R = 4096
_S = 50
_NCH = 2
_RC = _R // _NCH
_NROW = _RC * _S * _K
_ROW_PER_W = _NROW // _NW
_CHUNK = 128
_NCHUNK = _ROW_PER_W // _CHUNK
_BC = 128


def _tc_prep_body(tab_t_ref, out_ref):
    out_ref[...] = jnp.transpose(tab_t_ref[...], (1, 0))


def _tc_prep(tab_t):
    return pl.pallas_call(
        _tc_prep_body,
        grid=(_K, _VPAD // _VB),
        in_specs=[pl.BlockSpec((_L, _VB), lambda k, i: (k, i))],
        out_specs=pl.BlockSpec(
            (_VB, _L), lambda k, i: (k * (_VPAD // _VB) + i, 0)
        ),
        out_shape=jax.ShapeDtypeStruct((_K * _VPAD, _L), jnp.float32),
    )(tab_t)


def _sc_body(x4_hbm, tab4_hbm, mid_hbm, idx_v, buf0, buf1, sg0, sg1, sw0, sw1):
    wid = lax.axis_index("s") * _NC + lax.axis_index("c")
    base = wid * _ROW_PER_W
    pltpu.sync_copy(x4_hbm.at[pl.ds(base, _ROW_PER_W)], idx_v)

    bufs = (buf0, buf1)
    sgs = (sg0, sg1)
    sws = (sw0, sw1)

    @pl.loop(0, _NCHUNK, step=2)
    def _chunks(g):
        for b in range(2):
            cc = g + b

            @pl.when(cc >= 2)
            def _drain_write():
                pltpu.make_async_copy(
                    bufs[b], mid_hbm.at[pl.ds(base, _CHUNK)], sws[b]
                ).wait()

            pltpu.async_copy(
                tab4_hbm.at[idx_v.at[pl.ds(cc * _CHUNK, _CHUNK)]],
                bufs[b],
                sgs[b],
            )
            pltpu.make_async_copy(
                tab4_hbm.at[idx_v.at[pl.ds(0, _CHUNK)]], bufs[b], sgs[b]
            ).wait()
            pltpu.async_copy(
                bufs[b], mid_hbm.at[pl.ds(base + cc * _CHUNK, _CHUNK)], sws[b]
            )

    for b in range(2):
        pltpu.make_async_copy(
            bufs[b], mid_hbm.at[pl.ds(base, _CHUNK)], sws[b]
        ).wait()


def _sc_gather(x4, tab4):
    run = pl.kernel(
        _sc_body,
        out_type=jax.ShapeDtypeStruct((_NROW, _L), jnp.float32),
        mesh=plsc.VectorSubcoreMesh(core_axis_name="c", subcore_axis_name="s"),
        scratch_types=[
            pltpu.VMEM((_ROW_PER_W,), jnp.int32),
            pltpu.VMEM((_CHUNK, _L), jnp.float32),
            pltpu.VMEM((_CHUNK, _L), jnp.float32),
            pltpu.SemaphoreType.DMA,
            pltpu.SemaphoreType.DMA,
            pltpu.SemaphoreType.DMA,
            pltpu.SemaphoreType.DMA,
        ],
    )
    return run(x4, tab4)


def _tc_out_body(mid_ref, out_ref):
    x = mid_ref[...]
    for s in range(_S):
        for k in range(_K):
            t = jnp.transpose(x[:, _K * s + k, :], (1, 0))
            lo = _L * k
            hi = min(_L * (k + 1), _EMBED_DIM)
            out_ref[s, lo:hi, :] = t[: hi - lo, :]


def _tc_out_acc_body(mid_ref, acc_ref, out_ref):
    del acc_ref
    _tc_out_body(mid_ref, out_ref)


def _tc_out(mid4, c, acc):
    off = c * (_RC // _BC)
    out_shape = jax.ShapeDtypeStruct((_S, _EMBED_DIM, _R), jnp.float32)
    params = pltpu.CompilerParams(vmem_limit_bytes=56 * 1024 * 1024)
    mid_spec = pl.BlockSpec((_BC, _S * _K, _L), lambda i: (i, 0, 0))
    out_spec = pl.BlockSpec(
        (_S, _EMBED_DIM, _BC), lambda i, off=off: (0, 0, off + i)
    )
    if acc is None:
        return pl.pallas_call(
            _tc_out_body,
            grid=(_RC // _BC,),
            in_specs=[mid_spec],
            out_specs=out_spec,
            out_shape=out_shape,
            compiler_params=params,
        )(mid4)
    return pl.pallas_call(
        _tc_out_acc_body,
        grid=(_RC // _BC,),
        in_specs=[mid_spec, pl.BlockSpec(memory_space=pl.ANY)],
        out_specs=out_spec,
        out_shape=out_shape,
        input_output_aliases={1: 0},
        compiler_params=params,
    )(mid4, acc)


def kernel(x, table):
    tab4 = _tc_prep(jnp.transpose(table, (1, 0)))
    x_flat = x.reshape(-1).astype(jnp.int32)
    xr = jnp.repeat(x_flat.reshape(-1, _L // _K), _K, axis=1)
    kpat = jnp.tile(jnp.arange(_K, dtype=jnp.int32) * _VPAD, _L // _K)
    x4 = (xr + kpat[None, :]).reshape(-1)
    acc = None
    for c in range(_NCH):
        x4_c = lax.slice(x4, (c * _NROW,), ((c + 1) * _NROW,))
        mid = _sc_gather(x4_c, tab4)
        acc = _tc_out(mid.reshape(_RC, _S * _K, _L), c, acc)
    return jnp.transpose(acc, (2, 0, 1))
